# baseline (speedup 1.0000x reference)
import jax
import jax.numpy as jnp
from jax.experimental import pallas as pl
from jax.experimental.pallas import tpu as pltpu

E = 8
K = 2
T = 2048
D = 1024
F = 2816

BM = 256
NB = (T * K) // BM + E - 1
NP = NB * BM
BT = 256
BF = 704
NF = F // BF


def _gmm_body(be_ref, tr_ref, nxe_ref, hnx_ref, hs_hbm, w13_hbm, w2_hbm,
              tc_ref, y_ref, hs_scr, stg13, stg2, wb13, wb2,
              sem_h, sem13, sem2):
    b = pl.program_id(0)

    def start_weights(e_idx):
        pltpu.make_async_copy(w13_hbm.at[e_idx], stg13, sem13).start()
        pltpu.make_async_copy(w2_hbm.at[e_idx], stg2, sem2).start()

    def wait_weights():
        pltpu.make_async_copy(w13_hbm.at[0], stg13, sem13).wait()
        pltpu.make_async_copy(w2_hbm.at[0], stg2, sem2).wait()

    @pl.when(b == 0)
    def _init():
        cp = pltpu.make_async_copy(hs_hbm, hs_scr, sem_h)
        cp.start()
        start_weights(be_ref[0])
        cp.wait()

    @pl.when(tr_ref[b] == 1)
    def _transition():
        wait_weights()
        wb13[...] = stg13[...].astype(jnp.bfloat16)
        wb2[...] = stg2[...].astype(jnp.bfloat16)

        @pl.when(hnx_ref[b] == 1)
        def _prefetch():
            start_weights(nxe_ref[b])

    tc = tc_ref[0]
    tcol = tc[:, 0:1].astype(jnp.int32)
    cf = tc[:, 1:2]
    iota = jax.lax.broadcasted_iota(jnp.int32, (BM, T), 1)
    p = (iota == tcol).astype(jnp.bfloat16)
    a = jax.lax.dot_general(p, hs_scr[...], (((1,), (0,)), ((), ())),
                            preferred_element_type=jnp.float32)
    a = a.astype(jnp.bfloat16)

    part = jnp.zeros((BM, D), jnp.float32)
    for fi in range(NF):
        wg = wb13[fi * BF:(fi + 1) * BF, :]
        wu = wb13[F + fi * BF:F + (fi + 1) * BF, :]
        g = jax.lax.dot_general(a, wg, (((1,), (1,)), ((), ())),
                                preferred_element_type=jnp.float32)
        u = jax.lax.dot_general(a, wu, (((1,), (1,)), ((), ())),
                                preferred_element_type=jnp.float32)
        act = (g * jax.nn.sigmoid(g) * u).astype(jnp.bfloat16)
        w2c = wb2[:, fi * BF:(fi + 1) * BF]
        part = part + jax.lax.dot_general(
            act, w2c, (((1,), (1,)), ((), ())),
            preferred_element_type=jnp.float32)
    y_ref[...] = (part * cf).astype(jnp.bfloat16)


def _combine_body(y_hbm, tid_ref, out_ref, y_scr, sem_y):
    t = pl.program_id(0)

    @pl.when(t == 0)
    def _first():
        cp = pltpu.make_async_copy(y_hbm, y_scr, sem_y)
        cp.start()
        cp.wait()

    iota = jax.lax.broadcasted_iota(jnp.int32, (BT, NP), 0) + t * BT
    c = (iota == tid_ref[...]).astype(jnp.bfloat16)
    out_ref[...] = jax.lax.dot_general(
        c, y_scr[...], (((1,), (0,)), ((), ())),
        preferred_element_type=jnp.float32)


@jax.jit
def kernel(hidden_states, expert_routing_table, router_weights, w13, w2):
    TK = T * K
    eflat = expert_routing_table.reshape(TK)
    rw = router_weights.reshape(TK)
    tok = jnp.arange(TK, dtype=jnp.int32) // K

    onehot = (eflat[:, None] == jnp.arange(E, dtype=jnp.int32)[None, :])
    oh32 = onehot.astype(jnp.int32)
    incl = jnp.cumsum(oh32, axis=0)
    rank = jnp.sum(incl * oh32, axis=1) - 1
    counts = incl[-1]
    nblk = (counts + BM - 1) // BM
    ends = jnp.cumsum(nblk)
    starts = ends - nblk
    pos = starts[eflat] * BM + rank

    tid = jnp.full((NP,), -1, jnp.int32).at[pos].set(tok)
    coef = jnp.zeros((NP,), jnp.float32).at[pos].set(rw)

    bidx = jnp.arange(NB, dtype=jnp.int32)
    block_expert = jnp.minimum(
        jnp.sum((bidx[:, None] >= ends[None, :]).astype(jnp.int32), axis=1),
        E - 1)
    tr = jnp.concatenate([jnp.ones((1,), jnp.int32),
                          (block_expert[1:] != block_expert[:-1])
                          .astype(jnp.int32)])
    marks = jnp.where(tr == 1, bidx, NB)
    sufmin = jnp.flip(jax.lax.cummin(jnp.flip(
        jnp.concatenate([marks[1:], jnp.array([NB], jnp.int32)]))))
    hnx = (sufmin < NB).astype(jnp.int32)
    nxe = block_expert[jnp.clip(sufmin, 0, NB - 1)]

    hs16 = hidden_states.astype(jnp.bfloat16)
    tc = jnp.stack([tid.astype(jnp.float32), coef], axis=-1)
    tc3 = tc.reshape(NB, BM, 2)
    tid2 = tid.reshape(1, NP)

    gmm_spec = pltpu.PrefetchScalarGridSpec(
        num_scalar_prefetch=4,
        grid=(NB,),
        in_specs=[
            pl.BlockSpec(memory_space=pltpu.MemorySpace.HBM),
            pl.BlockSpec(memory_space=pltpu.MemorySpace.HBM),
            pl.BlockSpec(memory_space=pltpu.MemorySpace.HBM),
            pl.BlockSpec((1, BM, 2), lambda b, be, tr, nx, hn: (b, 0, 0)),
        ],
        out_specs=pl.BlockSpec((BM, D), lambda b, be, tr, nx, hn: (b, 0)),
        scratch_shapes=[
            pltpu.VMEM((T, D), jnp.bfloat16),
            pltpu.VMEM((2 * F, D), jnp.float32),
            pltpu.VMEM((D, F), jnp.float32),
            pltpu.VMEM((2 * F, D), jnp.bfloat16),
            pltpu.VMEM((D, F), jnp.bfloat16),
            pltpu.SemaphoreType.DMA,
            pltpu.SemaphoreType.DMA,
            pltpu.SemaphoreType.DMA,
        ],
    )

    y = pl.pallas_call(
        _gmm_body,
        grid_spec=gmm_spec,
        out_shape=jax.ShapeDtypeStruct((NP, D), jnp.bfloat16),
        compiler_params=pltpu.CompilerParams(
            dimension_semantics=("arbitrary",),
            vmem_limit_bytes=64 * 1024 * 1024,
        ),
    )(block_expert, tr, nxe, hnx, hs16, w13, w2, tc3)

    out = pl.pallas_call(
        _combine_body,
        grid=(T // BT,),
        in_specs=[
            pl.BlockSpec(memory_space=pltpu.MemorySpace.HBM),
            pl.BlockSpec((1, NP), lambda t: (0, 0)),
        ],
        out_specs=pl.BlockSpec((BT, D), lambda t: (t, 0)),
        out_shape=jax.ShapeDtypeStruct((T, D), jnp.float32),
        scratch_shapes=[
            pltpu.VMEM((NP, D), jnp.bfloat16),
            pltpu.SemaphoreType.DMA,
        ],
        compiler_params=pltpu.CompilerParams(
            dimension_semantics=("arbitrary",),
            vmem_limit_bytes=64 * 1024 * 1024,
        ),
    )(y, tid2)
    return out
